# baseline (speedup 1.0000x reference)
import jax
import jax.numpy as jnp
from jax import lax
from jax.experimental import pallas as pl
from jax.experimental.pallas import tpu as pltpu
from jax.experimental.pallas import tpu_sc as plsc

_D = 128
_SEQ = 200
_B = 1024
_NW = 32
_BPW = _B // _NW
_CH = _SEQ // 2
_NCHUNK = _BPW * 2
_NVEC = _D // 16


def _accum(rows_ref, init_vecs):
  def body(r, vecs):
    return tuple(vecs[c] + rows_ref[r, pl.ds(c * 16, 16)]
                 for c in range(_NVEC))
  return lax.fori_loop(0, _CH, body, init_vecs, unroll=4)


_NBUF = 8


def _pool_body(xt_hbm, table_hbm, out_hbm, idx_v, r0, r1, r2, r3, r4, r5, r6,
               r7, acc_v, s0, s1, s2, s3, s4, s5, s6, s7):
  rows = (r0, r1, r2, r3, r4, r5, r6, r7)
  sems = (s0, s1, s2, s3, s4, s5, s6, s7)
  wid = lax.axis_index("s") * 2 + lax.axis_index("c")
  base = wid * _NCHUNK
  pltpu.sync_copy(xt_hbm.at[pl.ds(base, _NCHUNK)], idx_v)
  for k in range(_NBUF - 1):
    pltpu.async_copy(table_hbm.at[idx_v.at[k]], rows[k], sems[k])

  zeros = tuple(jnp.zeros((16,), jnp.float32) for _ in range(_NVEC))

  def ring_step(g, n_issue):
    vecs = zeros
    for k in range(_NBUF):
      j = _NBUF * g + k
      if k < n_issue:
        nb = (k + _NBUF - 1) % _NBUF
        pltpu.async_copy(table_hbm.at[idx_v.at[j + _NBUF - 1]], rows[nb],
                         sems[nb])
      pltpu.make_async_copy(table_hbm.at[idx_v.at[j]], rows[k],
                            sems[k]).wait()
      vecs = _accum(rows[k], vecs)
      if k % 2 == 1:
        for c in range(_NVEC):
          acc_v[(_NBUF // 2) * g + k // 2, pl.ds(c * 16, 16)] = vecs[c]
        vecs = zeros

  def body(g, carry):
    ring_step(g, n_issue=_NBUF)
    return carry

  n_full = (_NCHUNK - (_NBUF - 1)) // _NBUF
  lax.fori_loop(0, n_full, body, 0)
  ring_step(n_full, n_issue=_NCHUNK - (_NBUF - 1) - _NBUF * n_full)
  pltpu.sync_copy(acc_v, out_hbm.at[pl.ds(wid * _BPW, _BPW)])


def _fc_body(p_ref, w_ref, b_ref, o_ref):
  o_ref[...] = (
      lax.dot_general(p_ref[...], w_ref[...], (((1,), (1,)), ((), ())),
                      preferred_element_type=jnp.float32) * (1.0 / _SEQ)
      + b_ref[...])


@jax.jit
def kernel(x, table, W, b):
  xt = jnp.transpose(x).reshape(_B * 2, _CH)
  pooled = pl.kernel(
      _pool_body,
      out_type=jax.ShapeDtypeStruct((_B, _D), jnp.float32),
      mesh=plsc.VectorSubcoreMesh(core_axis_name="c", subcore_axis_name="s"),
      scratch_types=(
          [pltpu.VMEM((_NCHUNK, _CH), jnp.int32)]
          + [pltpu.VMEM((_CH, _D), jnp.float32) for _ in range(_NBUF)]
          + [pltpu.VMEM((_BPW, _D), jnp.float32)]
          + [pltpu.SemaphoreType.DMA for _ in range(_NBUF)]
      ),
  )(xt, table)
  return pl.pallas_call(
      _fc_body,
      out_shape=jax.ShapeDtypeStruct((_B, _D), jnp.float32),
  )(pooled, W, b.reshape(1, _D))


# speedup vs baseline: 1.0172x; 1.0172x over previous
import jax
import jax.numpy as jnp
from jax import lax
from jax.experimental import pallas as pl
from jax.experimental.pallas import tpu as pltpu
from jax.experimental.pallas import tpu_sc as plsc

_D = 128
_SEQ = 200
_B = 1024
_NW = 32
_BPW = _B // _NW
_CH = _SEQ // 2
_NCHUNK = _BPW * 2
_NVEC = _D // 16


def _accum(rows_ref, init_vecs):
  def body(r, vecs):
    return tuple(vecs[c] + rows_ref[r, pl.ds(c * 16, 16)]
                 for c in range(_NVEC))
  return lax.fori_loop(0, _CH, body, init_vecs, unroll=10)


_NBUF = 4


def _pool_body(xt_hbm, table_hbm, out_hbm, idx_v, r0, r1, r2, r3, acc_v,
               s0, s1, s2, s3):
  rows = (r0, r1, r2, r3)
  sems = (s0, s1, s2, s3)
  wid = lax.axis_index("s") * 2 + lax.axis_index("c")
  base = wid * _NCHUNK
  pltpu.sync_copy(xt_hbm.at[pl.ds(base, _NCHUNK)], idx_v)
  for k in range(_NBUF - 1):
    pltpu.async_copy(table_hbm.at[idx_v.at[k]], rows[k], sems[k])

  zeros = tuple(jnp.zeros((16,), jnp.float32) for _ in range(_NVEC))

  def ring_step(g, n_issue):
    vecs = zeros
    for k in range(_NBUF):
      j = _NBUF * g + k
      if k < n_issue:
        nb = (k + _NBUF - 1) % _NBUF
        pltpu.async_copy(table_hbm.at[idx_v.at[j + _NBUF - 1]], rows[nb],
                         sems[nb])
      pltpu.make_async_copy(table_hbm.at[idx_v.at[j]], rows[k],
                            sems[k]).wait()
      vecs = _accum(rows[k], vecs)
      if k % 2 == 1:
        for c in range(_NVEC):
          acc_v[(_NBUF // 2) * g + k // 2, pl.ds(c * 16, 16)] = vecs[c]
        vecs = zeros

  def body(g, carry):
    ring_step(g, n_issue=_NBUF)
    return carry

  n_full = (_NCHUNK - (_NBUF - 1)) // _NBUF
  lax.fori_loop(0, n_full, body, 0)
  ring_step(n_full, n_issue=_NCHUNK - (_NBUF - 1) - _NBUF * n_full)
  pltpu.sync_copy(acc_v, out_hbm.at[pl.ds(wid * _BPW, _BPW)])


def _fc_body(p_ref, w_ref, b_ref, o_ref):
  o_ref[...] = (
      lax.dot_general(p_ref[...], w_ref[...], (((1,), (1,)), ((), ())),
                      preferred_element_type=jnp.float32) * (1.0 / _SEQ)
      + b_ref[...])


@jax.jit
def kernel(x, table, W, b):
  xt = jnp.transpose(x).reshape(_B * 2, _CH)
  pooled = pl.kernel(
      _pool_body,
      out_type=jax.ShapeDtypeStruct((_B, _D), jnp.float32),
      mesh=plsc.VectorSubcoreMesh(core_axis_name="c", subcore_axis_name="s"),
      scratch_types=(
          [pltpu.VMEM((_NCHUNK, _CH), jnp.int32)]
          + [pltpu.VMEM((_CH, _D), jnp.float32) for _ in range(_NBUF)]
          + [pltpu.VMEM((_BPW, _D), jnp.float32)]
          + [pltpu.SemaphoreType.DMA for _ in range(_NBUF)]
      ),
  )(xt, table)
  return pl.pallas_call(
      _fc_body,
      out_shape=jax.ShapeDtypeStruct((_B, _D), jnp.float32),
  )(pooled, W, b.reshape(1, _D))


# speedup vs baseline: 1.0438x; 1.0261x over previous
import jax
import jax.numpy as jnp
from jax import lax
from jax.experimental import pallas as pl
from jax.experimental.pallas import tpu as pltpu
from jax.experimental.pallas import tpu_sc as plsc

_D = 128
_SEQ = 200
_B = 1024
_NW = 32
_BPW = _B // _NW
_CH = _SEQ // 2
_NCHUNK = _BPW * 2
_NVEC = _D // 16


def _accum(rows_ref, init_vecs):
  def body(r, vecs):
    return tuple(vecs[c] + rows_ref[r, pl.ds(c * 16, 16)]
                 for c in range(_NVEC))
  return lax.fori_loop(0, _CH, body, init_vecs, unroll=4)


_NBUF = 4


def _pool_body(xt_hbm, table_hbm, out_hbm, idx_v, r0, r1, r2, r3, acc_v,
               s0, s1, s2, s3):
  rows = (r0, r1, r2, r3)
  sems = (s0, s1, s2, s3)
  wid = lax.axis_index("s") * 2 + lax.axis_index("c")
  pltpu.sync_copy(xt_hbm.at[pl.ds(wid * _BPW, _BPW)], idx_v)

  def idx_of(j, k):
    return idx_v.at[j // 2, k % 2]

  for k in range(_NBUF - 1):
    pltpu.async_copy(table_hbm.at[idx_of(k, k)], rows[k], sems[k])

  zeros = tuple(jnp.zeros((16,), jnp.float32) for _ in range(_NVEC))

  def ring_step(g, n_issue):
    vecs = zeros
    for k in range(_NBUF):
      j = _NBUF * g + k
      if k < n_issue:
        nb = (k + _NBUF - 1) % _NBUF
        pltpu.async_copy(table_hbm.at[idx_of(j + _NBUF - 1, k + _NBUF - 1)],
                         rows[nb], sems[nb])
      pltpu.make_async_copy(table_hbm.at[idx_of(j, k)], rows[k],
                            sems[k]).wait()
      vecs = _accum(rows[k], vecs)
      if k % 2 == 1:
        for c in range(_NVEC):
          acc_v[(_NBUF // 2) * g + k // 2, pl.ds(c * 16, 16)] = vecs[c]
        vecs = zeros

  def body(g, carry):
    ring_step(g, n_issue=_NBUF)
    return carry

  n_full = (_NCHUNK - (_NBUF - 1)) // _NBUF
  lax.fori_loop(0, n_full, body, 0)
  ring_step(n_full, n_issue=_NCHUNK - (_NBUF - 1) - _NBUF * n_full)
  pltpu.sync_copy(acc_v, out_hbm.at[pl.ds(wid * _BPW, _BPW)])


def _fc_body(p_ref, w_ref, b_ref, o_ref):
  o_ref[...] = (
      lax.dot_general(p_ref[...], w_ref[...], (((1,), (1,)), ((), ())),
                      preferred_element_type=jnp.float32) * (1.0 / _SEQ)
      + b_ref[...])


@jax.jit
def kernel(x, table, W, b):
  xt = jnp.transpose(x).reshape(_B, 2, _CH)
  pooled = pl.kernel(
      _pool_body,
      out_type=jax.ShapeDtypeStruct((_B, _D), jnp.float32),
      mesh=plsc.VectorSubcoreMesh(core_axis_name="c", subcore_axis_name="s"),
      scratch_types=(
          [pltpu.VMEM((_BPW, 2, _CH), jnp.int32)]
          + [pltpu.VMEM((_CH, _D), jnp.float32) for _ in range(_NBUF)]
          + [pltpu.VMEM((_BPW, _D), jnp.float32)]
          + [pltpu.SemaphoreType.DMA for _ in range(_NBUF)]
      ),
  )(xt, table)
  return pl.pallas_call(
      _fc_body,
      out_shape=jax.ShapeDtypeStruct((_B, _D), jnp.float32),
  )(pooled, W, b.reshape(1, _D))


# speedup vs baseline: 1.0490x; 1.0051x over previous
import jax
import jax.numpy as jnp
from jax import lax
from jax.experimental import pallas as pl
from jax.experimental.pallas import tpu as pltpu
from jax.experimental.pallas import tpu_sc as plsc

_D = 128
_SEQ = 200
_B = 1024
_NW = 32
_BPW = _B // _NW
_CH = _SEQ // 2
_NCHUNK = _BPW * 2
_NVEC = _D // 16


def _accum(rows_ref, init_vecs):
  def body(r, vecs):
    return tuple(vecs[c] + rows_ref[r, pl.ds(c * 16, 16)]
                 for c in range(_NVEC))
  return lax.fori_loop(0, _CH, body, init_vecs, unroll=4)


_NBUF = 4


def _pool_body(xt_hbm, table_hbm, out_hbm, idx_v, r0, r1, r2, r3, acc_v,
               s0, s1, s2, s3):
  rows = (r0, r1, r2, r3)
  sems = (s0, s1, s2, s3)
  wid = lax.axis_index("s") * 2 + lax.axis_index("c")
  pltpu.sync_copy(xt_hbm.at[pl.ds(wid * _BPW, _BPW)], idx_v)

  def idx_of(j, k):
    return idx_v.at[j // 2, k % 2]

  for k in range(_NBUF - 1):
    pltpu.async_copy(table_hbm.at[idx_of(k, k)], rows[k], sems[k])

  zeros = tuple(jnp.zeros((16,), jnp.float32) for _ in range(_NVEC))

  def ring_step(g, n_issue):
    vecs = zeros
    for k in range(_NBUF):
      j = _NBUF * g + k
      if k < n_issue:
        nb = (k + _NBUF - 1) % _NBUF
        pltpu.async_copy(table_hbm.at[idx_of(j + _NBUF - 1, k + _NBUF - 1)],
                         rows[nb], sems[nb])
      pltpu.make_async_copy(table_hbm.at[idx_of(j, k)], rows[k],
                            sems[k]).wait()
      vecs = _accum(rows[k], vecs)
      if k % 2 == 1:
        for c in range(_NVEC):
          acc_v[(_NBUF // 2) * g + k // 2, pl.ds(c * 16, 16)] = vecs[c]
        vecs = zeros

  def body(g, carry):
    ring_step(g, n_issue=_NBUF)
    return carry

  n_full = (_NCHUNK - (_NBUF - 1)) // _NBUF
  lax.fori_loop(0, n_full, body, 0)
  ring_step(n_full, n_issue=_NCHUNK - (_NBUF - 1) - _NBUF * n_full)
  pltpu.sync_copy(acc_v, out_hbm.at[pl.ds(wid * _BPW, _BPW)])


def _fc_body(p_ref, w_ref, b_ref, o_ref):
  o_ref[...] = (
      lax.dot_general(p_ref[...], w_ref[...], (((1,), (1,)), ((), ())),
                      preferred_element_type=jnp.float32) * (1.0 / _SEQ)
      + b_ref[...])


@jax.jit
def kernel(x, table, W, b):
  xt = jnp.transpose(x.reshape(2, _CH, _B), (2, 0, 1))
  pooled = pl.kernel(
      _pool_body,
      out_type=jax.ShapeDtypeStruct((_B, _D), jnp.float32),
      mesh=plsc.VectorSubcoreMesh(core_axis_name="c", subcore_axis_name="s"),
      scratch_types=(
          [pltpu.VMEM((_BPW, 2, _CH), jnp.int32)]
          + [pltpu.VMEM((_CH, _D), jnp.float32) for _ in range(_NBUF)]
          + [pltpu.VMEM((_BPW, _D), jnp.float32)]
          + [pltpu.SemaphoreType.DMA for _ in range(_NBUF)]
      ),
  )(xt, table)
  return pl.pallas_call(
      _fc_body,
      out_shape=jax.ShapeDtypeStruct((_B, _D), jnp.float32),
  )(pooled, W, b.reshape(1, _D))
